# Initial kernel scaffold; baseline (speedup 1.0000x reference)
import jax
import jax.numpy as jnp
from jax.experimental import pallas as pl


def kernel(edge_index, edge_weight, ini_embeds):
    raise NotImplementedError("write your pallas kernel here")


# speedup vs baseline: 3.7704x; 3.7704x over previous
import functools

import jax
import jax.numpy as jnp
from jax import lax
from jax.experimental import pallas as pl
from jax.experimental.pallas import tpu as pltpu
from jax.experimental.pallas import tpu_sc as plsc

LAT = 128
NLAYER = 3
LANES = 16
_C = 80


def _sc_layer_fn(N, E):
    info = plsc.get_sparse_core_info()
    NC, NS = info.num_cores, info.num_subcores
    NW = NC * NS
    ept = E // NW
    zrows = 40
    nblk = N // zrows
    mesh = plsc.VectorSubcoreMesh(core_axis_name="c", subcore_axis_name="s")

    @functools.partial(
        pl.kernel,
        out_type=jax.ShapeDtypeStruct((NC, N, LAT), jnp.float32),
        mesh=mesh,
        scratch_types=[
            pltpu.VMEM((_C,), jnp.int32),
            pltpu.VMEM((_C,), jnp.int32),
            pltpu.VMEM((_C,), jnp.float32),
            pltpu.VMEM((_C, LAT), jnp.float32),
            pltpu.VMEM((zrows, LAT), jnp.float32),
            pltpu.VMEM_SHARED((N, LAT), jnp.float32),
            pltpu.SemaphoreType.DMA,
        ],
    )
    def k(src_h, dst_h, w_h, x_h, out_h, srcv, dstv, wv, msgs, zbuf, acc, sem):
        cid = lax.axis_index("c")
        sid = lax.axis_index("s")
        wid = sid * NC + cid

        def zfill(r, _):
            for j in range(LAT // LANES):
                zbuf[r, pl.ds(LANES * j, LANES)] = jnp.zeros((LANES,), jnp.float32)
            return _
        lax.fori_loop(0, zrows, zfill, None)

        def zcopy(t, _):
            blk = t * NS + sid
            @pl.when(blk < nblk)
            def _do():
                pltpu.sync_copy(zbuf, acc.at[pl.ds(blk * zrows, zrows)])
            return _
        lax.fori_loop(0, pl.cdiv(nblk, NS), zcopy, None)
        plsc.subcore_barrier()

        base0 = wid * ept

        def chunk(i, _):
            b = base0 + i * _C
            pltpu.sync_copy(src_h.at[pl.ds(b, _C)], srcv)
            pltpu.sync_copy(dst_h.at[pl.ds(b, _C)], dstv)
            pltpu.sync_copy(w_h.at[pl.ds(b, _C)], wv)
            pltpu.async_copy(x_h.at[srcv], msgs, sem).wait()

            def scale(g, _):
                wvec = wv[pl.ds(g * LANES, LANES)]
                for t in range(LANES):
                    we = wvec[t]
                    e = g * LANES + t
                    for j in range(LAT // LANES):
                        msgs[e, pl.ds(LANES * j, LANES)] = (
                            msgs[e, pl.ds(LANES * j, LANES)] * we)
                return _
            lax.fori_loop(0, _C // LANES, scale, None)
            pltpu.sync_copy(msgs, acc.at[dstv], add=True)
            return _
        lax.fori_loop(0, ept // _C, chunk, None)
        plsc.subcore_barrier()

        def wb(t, _):
            blk = t * NS + sid
            @pl.when(blk < nblk)
            def _do():
                pltpu.sync_copy(acc.at[pl.ds(blk * zrows, zrows)],
                                out_h.at[cid, pl.ds(blk * zrows, zrows)])
            return _
        lax.fori_loop(0, pl.cdiv(nblk, NS), wb, None)

    return k


def _merge(p, runsum):
    N, _ = runsum.shape
    blk = 400

    def mk(p_ref, rs_ref, x_ref, rs2_ref):
        a = p_ref[0] + p_ref[1]
        x_ref[...] = a
        rs2_ref[...] = rs_ref[...] + a

    return pl.pallas_call(
        mk,
        grid=(N // blk,),
        in_specs=[
            pl.BlockSpec((2, blk, LAT), lambda i: (0, i, 0)),
            pl.BlockSpec((blk, LAT), lambda i: (i, 0)),
        ],
        out_specs=[
            pl.BlockSpec((blk, LAT), lambda i: (i, 0)),
            pl.BlockSpec((blk, LAT), lambda i: (i, 0)),
        ],
        out_shape=[jax.ShapeDtypeStruct((N, LAT), jnp.float32)] * 2,
    )(p, runsum)


def kernel(edge_index, edge_weight, ini_embeds):
    src = edge_index[0]
    dst = edge_index[1]
    N = ini_embeds.shape[0]
    E = edge_weight.shape[0]
    layer = _sc_layer_fn(N, E)
    x = ini_embeds
    runsum = ini_embeds
    for _ in range(NLAYER):
        p = layer(src, dst, edge_weight, x)
        x, runsum = _merge(p, runsum)
    half = N // 2
    return runsum[:half], runsum[half:]


# speedup vs baseline: 9.5502x; 2.5330x over previous
import functools

import jax
import jax.numpy as jnp
from jax import lax
from jax.experimental import pallas as pl
from jax.experimental.pallas import tpu as pltpu
from jax.experimental.pallas import tpu_sc as plsc

LAT = 128
NLAYER = 3
LANES = 16
_C = 80
_BCH = 25
_NB = 5


def _sc_layer_fn(N, E):
    info = plsc.get_sparse_core_info()
    NC, NS = info.num_cores, info.num_subcores
    NW = NC * NS
    ept = E // NW
    assert ept == _NB * _BCH * _C
    nzb = N // _C
    mesh = plsc.VectorSubcoreMesh(core_axis_name="c", subcore_axis_name="s")

    @functools.partial(
        pl.kernel,
        out_type=jax.ShapeDtypeStruct((NC, N, LAT), jnp.float32),
        mesh=mesh,
        scratch_types=[
            pltpu.VMEM((2, _BCH, _C), jnp.int32),
            pltpu.VMEM((2, _BCH, _C), jnp.int32),
            pltpu.VMEM((2, _BCH, _C), jnp.float32),
            pltpu.VMEM((_C, LAT), jnp.float32),
            pltpu.VMEM((_C, LAT), jnp.float32),
            pltpu.VMEM_SHARED((N, LAT), jnp.float32),
            pltpu.SemaphoreType.DMA,
            pltpu.SemaphoreType.DMA,
            pltpu.SemaphoreType.DMA,
            pltpu.SemaphoreType.DMA,
        ],
    )
    def k(src_h, dst_h, w_h, x_h, out_h,
          srcb, dstb, wb, m0, m1, acc, si0, si1, sg0, sg1):
        cid = lax.axis_index("c")
        sid = lax.axis_index("s")
        wid = sid * NC + cid
        M = (m0, m1)
        SG = (sg0, sg1)
        SI = (si0, si1)

        def fire_block(b, par):
            return [
                pltpu.async_copy(src_h.at[wid, b], srcb.at[par], SI[par]),
                pltpu.async_copy(dst_h.at[wid, b], dstb.at[par], SI[par]),
                pltpu.async_copy(w_h.at[wid, b], wb.at[par], SI[par]),
            ]

        def gather(par_blk, c, par_msg):
            return pltpu.async_copy(
                x_h.at[srcb.at[par_blk, c]], M[par_msg], SG[par_msg])

        def gather_wait(par_blk, c, par_msg):
            pltpu.make_async_copy(
                x_h.at[srcb.at[par_blk, c]], M[par_msg], SG[par_msg]).wait()

        def scale(mref, par_blk, c):
            def sgrp(g, _):
                wvec = wb[par_blk, c, pl.ds(g * LANES, LANES)]
                for t in range(LANES):
                    we = wvec[t]
                    e = g * LANES + t
                    for j in range(LAT // LANES):
                        mref[e, pl.ds(LANES * j, LANES)] = (
                            mref[e, pl.ds(LANES * j, LANES)] * we)
                return _
            lax.fori_loop(0, _C // LANES, sgrp, None)

        def scatter(mref, par_blk, c):
            pltpu.sync_copy(mref, acc.at[dstb.at[par_blk, c]], add=True)

        h0 = fire_block(0, 0)

        def zfill(r, _):
            for j in range(LAT // LANES):
                m0[r, pl.ds(LANES * j, LANES)] = jnp.zeros((LANES,), jnp.float32)
            return _
        lax.fori_loop(0, _C, zfill, None)

        def zcopy(t, _):
            blk = t * NS + sid
            @pl.when(blk < nzb)
            def _do():
                pltpu.sync_copy(m0, acc.at[pl.ds(blk * _C, _C)])
            return _
        lax.fori_loop(0, pl.cdiv(nzb, NS), zcopy, None)
        plsc.subcore_barrier()

        for h in h0:
            h.wait()
        gather(0, 0, 0)

        for b in range(_NB):
            p = b % 2
            np_ = 1 - p
            hnext = fire_block(b + 1, np_) if b + 1 < _NB else None

            def body2(g, _, p=p, np_=np_):
                for dc, q in ((0, p), (1, np_)):
                    c = 2 * g + dc
                    qn = 1 - q
                    gather(p, c + 1, qn)
                    gather_wait(p, c, q)
                    scale(M[q], p, c)
                    scatter(M[q], p, c)
                return _
            lax.fori_loop(0, (_BCH - 1) // 2, body2, None)

            if hnext is not None:
                for h in hnext:
                    h.wait()
                gather(np_, 0, np_)
            gather_wait(p, _BCH - 1, p)
            scale(M[p], p, _BCH - 1)
            scatter(M[p], p, _BCH - 1)

        plsc.subcore_barrier()

        def wb_loop(t, _):
            blk = t * NS + sid
            @pl.when(blk < nzb)
            def _do():
                pltpu.sync_copy(acc.at[pl.ds(blk * _C, _C)],
                                out_h.at[cid, pl.ds(blk * _C, _C)])
            return _
        lax.fori_loop(0, pl.cdiv(nzb, NS), wb_loop, None)

    return k


def _merge(p, runsum):
    N, _ = runsum.shape
    blk = 400

    def mk(p_ref, rs_ref, x_ref, rs2_ref):
        a = p_ref[0] + p_ref[1]
        x_ref[...] = a
        rs2_ref[...] = rs_ref[...] + a

    return pl.pallas_call(
        mk,
        grid=(N // blk,),
        in_specs=[
            pl.BlockSpec((2, blk, LAT), lambda i: (0, i, 0)),
            pl.BlockSpec((blk, LAT), lambda i: (i, 0)),
        ],
        out_specs=[
            pl.BlockSpec((blk, LAT), lambda i: (i, 0)),
            pl.BlockSpec((blk, LAT), lambda i: (i, 0)),
        ],
        out_shape=[jax.ShapeDtypeStruct((N, LAT), jnp.float32)] * 2,
    )(p, runsum)


def kernel(edge_index, edge_weight, ini_embeds):
    N = ini_embeds.shape[0]
    E = edge_weight.shape[0]
    info = plsc.get_sparse_core_info()
    NW = info.num_cores * info.num_subcores
    src = edge_index[0].reshape(NW, _NB, _BCH, _C)
    dst = edge_index[1].reshape(NW, _NB, _BCH, _C)
    w = edge_weight.reshape(NW, _NB, _BCH, _C)
    layer = _sc_layer_fn(N, E)
    x = ini_embeds
    runsum = ini_embeds
    for _ in range(NLAYER):
        part = layer(src, dst, w, x)
        x, runsum = _merge(part, runsum)
    half = N // 2
    return runsum[:half], runsum[half:]
